# baseline (speedup 1.0000x reference)
import jax
import jax.numpy as jnp
from jax import lax
from jax.experimental import pallas as pl
from jax.experimental.pallas import tpu as pltpu
from jax.experimental.pallas import tpu_sc as plsc

N = 20000
CPW = 640
CHUNKS = CPW // 16

_LN2 = 0.6931471805599453
_SQRT2 = 1.4142135623730951


def _flog(x):
    xi = lax.bitcast_convert_type(x, jnp.int32)
    m = lax.bitcast_convert_type(
        (xi & jnp.int32(0x007FFFFF)) | jnp.int32(0x3F800000), jnp.float32
    )
    e = (lax.shift_right_logical(xi, jnp.int32(23)) & jnp.int32(0xFF)) - 127
    big = m > _SQRT2
    m = jnp.where(big, m * 0.5, m)
    e = jnp.where(big, e + 1, e)
    s = (m - 1.0) / (m + 1.0)
    z = s * s
    p = 2.0 * s * (1.0 + z * (1.0 / 3.0 + z * (0.2 + z * (1.0 / 7.0 + z / 9.0))))
    return e.astype(jnp.float32) * _LN2 + p


def _sc_body(rois_h, gt_h, lab_h, tgt_h, w_h, w2_h, rois_v, gt_v, lab_v, tgt_v, w_v):
    wid = lax.axis_index("s") * 2 + lax.axis_index("c")

    def span(base, cols):
        base = pl.multiple_of(jnp.minimum(base, 19456), 128)
        pltpu.sync_copy(
            rois_h.at[pl.ds(1, 4), :, pl.ds(base, cols)],
            rois_v.at[:, :, pl.ds(0, cols)],
        )
        pltpu.sync_copy(
            gt_h.at[:, :, pl.ds(base, cols)], gt_v.at[:, :, pl.ds(0, cols)]
        )
        pltpu.sync_copy(lab_h.at[pl.ds(base, cols)], lab_v.at[pl.ds(0, cols)])

        def chunk(j, carry):
            sl = pl.ds(j * 16, 16)
            x1 = rois_v[0, 0, sl]
            y1 = rois_v[1, 0, sl]
            x2 = rois_v[2, 0, sl]
            y2 = rois_v[3, 0, sl]
            gx1 = gt_v[0, 0, sl]
            gy1 = gt_v[0, 1, sl]
            gx2 = gt_v[0, 2, sl]
            gy2 = gt_v[0, 3, sl]
            lab = lab_v[sl]

            ew = x2 - x1 + 1.0
            eh = y2 - y1 + 1.0
            gw = gx2 - gx1 + 1.0
            gh = gy2 - gy1 + 1.0
            dx = ((gx1 + 0.5 * gw) - (x1 + 0.5 * ew)) / ew * 10.0
            dy = ((gy1 + 0.5 * gh) - (y1 + 0.5 * eh)) / eh * 10.0
            dw = _flog(gw / ew) * 5.0
            dh = _flog(gh / eh) * 5.0

            fg = lab > 0
            zero = jnp.zeros_like(dx)
            wv = jnp.where(fg, zero + 1.0, zero)
            tgt_v[0, 0, sl] = jnp.where(fg, dx, zero)
            tgt_v[0, 1, sl] = jnp.where(fg, dy, zero)
            tgt_v[0, 2, sl] = jnp.where(fg, dw, zero)
            tgt_v[0, 3, sl] = jnp.where(fg, dh, zero)
            w_v[0, 0, sl] = wv
            w_v[0, 1, sl] = wv
            w_v[0, 2, sl] = wv
            w_v[0, 3, sl] = wv
            return carry

        lax.fori_loop(0, cols // 16, chunk, 0)

        pltpu.sync_copy(
            tgt_v.at[:, :, pl.ds(0, cols)], tgt_h.at[:, :, pl.ds(base, cols)]
        )
        pltpu.sync_copy(
            w_v.at[:, :, pl.ds(0, cols)], w_h.at[:, :, pl.ds(base, cols)]
        )
        pltpu.sync_copy(
            w_v.at[:, :, pl.ds(0, cols)], w2_h.at[:, :, pl.ds(base, cols)]
        )

    span(wid * CPW, CPW)


_sc_call = pl.kernel(
    _sc_body,
    out_type=(
        jax.ShapeDtypeStruct((1, 4, N), jnp.float32),
        jax.ShapeDtypeStruct((1, 4, N), jnp.float32),
        jax.ShapeDtypeStruct((1, 4, N), jnp.float32),
    ),
    mesh=plsc.VectorSubcoreMesh(core_axis_name="c", subcore_axis_name="s"),
    compiler_params=pltpu.CompilerParams(
        needs_layout_passes=False,
        skip_device_barrier=True,
        disable_bounds_checks=True,
        disable_semaphore_checks=True,
    ),
    scratch_types=[
        pltpu.VMEM((4, 1, CPW), jnp.float32),
        pltpu.VMEM((1, 4, CPW), jnp.float32),
        pltpu.VMEM((CPW,), jnp.int32),
        pltpu.VMEM((1, 4, CPW), jnp.float32),
        pltpu.VMEM((1, 4, CPW), jnp.float32),
    ],
)


@jax.jit
def kernel(gt_rois, rois, labels):
    rois_t = jnp.transpose(rois, (2, 0, 1))
    gt_t = jnp.transpose(gt_rois, (0, 2, 1))
    tgt_t, w_t, w2_t = _sc_call(rois_t, gt_t, labels)
    tgt = jnp.transpose(tgt_t, (0, 2, 1))
    w = jnp.transpose(w_t, (0, 2, 1))
    w2 = jnp.transpose(w2_t, (0, 2, 1))
    return tgt, w, w2


# speedup vs baseline: 3.7149x; 3.7149x over previous
import jax
import jax.numpy as jnp
from jax.experimental import pallas as pl
from jax.experimental.pallas import tpu as pltpu

N = 20000
BLK = 2048
GRID = (N + BLK - 1) // BLK


def _tc_body(rois_ref, gt_ref, lab_ref, tgt_ref, w_ref, w2_ref):
    x1 = rois_ref[1, 0, :]
    y1 = rois_ref[2, 0, :]
    x2 = rois_ref[3, 0, :]
    y2 = rois_ref[4, 0, :]
    gx1 = gt_ref[0, 0, :]
    gy1 = gt_ref[0, 1, :]
    gx2 = gt_ref[0, 2, :]
    gy2 = gt_ref[0, 3, :]
    lab = lab_ref[:]

    ew = x2 - x1 + 1.0
    eh = y2 - y1 + 1.0
    gw = gx2 - gx1 + 1.0
    gh = gy2 - gy1 + 1.0
    dx = ((gx1 + 0.5 * gw) - (x1 + 0.5 * ew)) / ew * 10.0
    dy = ((gy1 + 0.5 * gh) - (y1 + 0.5 * eh)) / eh * 10.0
    dw = jnp.log(gw / ew) * 5.0
    dh = jnp.log(gh / eh) * 5.0

    fg = lab > 0
    zero = jnp.zeros_like(dx)
    wv = jnp.where(fg, zero + 1.0, zero)
    tgt_ref[0, 0, :] = jnp.where(fg, dx, zero)
    tgt_ref[0, 1, :] = jnp.where(fg, dy, zero)
    tgt_ref[0, 2, :] = jnp.where(fg, dw, zero)
    tgt_ref[0, 3, :] = jnp.where(fg, dh, zero)
    w_ref[0, 0, :] = wv
    w_ref[0, 1, :] = wv
    w_ref[0, 2, :] = wv
    w_ref[0, 3, :] = wv
    w2_ref[0, 0, :] = wv
    w2_ref[0, 1, :] = wv
    w2_ref[0, 2, :] = wv
    w2_ref[0, 3, :] = wv


_out_bs = pl.BlockSpec((1, 4, BLK), lambda i: (0, 0, i))

_tc_call = pl.pallas_call(
    _tc_body,
    grid=(GRID,),
    in_specs=[
        pl.BlockSpec((5, 1, BLK), lambda i: (0, 0, i)),
        pl.BlockSpec((1, 4, BLK), lambda i: (0, 0, i)),
        pl.BlockSpec((BLK,), lambda i: (i,)),
    ],
    out_specs=[_out_bs, _out_bs, _out_bs],
    out_shape=(
        jax.ShapeDtypeStruct((1, 4, N), jnp.float32),
        jax.ShapeDtypeStruct((1, 4, N), jnp.float32),
        jax.ShapeDtypeStruct((1, 4, N), jnp.float32),
    ),
)


@jax.jit
def kernel(gt_rois, rois, labels):
    rois_t = jnp.transpose(rois, (2, 0, 1))
    gt_t = jnp.transpose(gt_rois, (0, 2, 1))
    tgt_t, w_t, w2_t = _tc_call(rois_t, gt_t, labels)
    tgt = jnp.transpose(tgt_t, (0, 2, 1))
    w = jnp.transpose(w_t, (0, 2, 1))
    w2 = jnp.transpose(w2_t, (0, 2, 1))
    return tgt, w, w2


# speedup vs baseline: 9.2648x; 2.4939x over previous
import jax
import jax.numpy as jnp
from jax.experimental import pallas as pl
from jax.experimental.pallas import tpu as pltpu

N = 20000
BLK = 12288
GRID = (N + BLK - 1) // BLK


def _tc_body(rois_ref, gt_ref, lab_ref, tgt_ref, w_ref, w2_ref):
    x1 = rois_ref[1, 0, :]
    y1 = rois_ref[2, 0, :]
    x2 = rois_ref[3, 0, :]
    y2 = rois_ref[4, 0, :]
    gx1 = gt_ref[0, 0, :]
    gy1 = gt_ref[0, 1, :]
    gx2 = gt_ref[0, 2, :]
    gy2 = gt_ref[0, 3, :]
    lab = lab_ref[:]

    ew = x2 - x1 + 1.0
    eh = y2 - y1 + 1.0
    gw = gx2 - gx1 + 1.0
    gh = gy2 - gy1 + 1.0
    dx = ((gx1 + 0.5 * gw) - (x1 + 0.5 * ew)) / ew * 10.0
    dy = ((gy1 + 0.5 * gh) - (y1 + 0.5 * eh)) / eh * 10.0
    dw = jnp.log(gw / ew) * 5.0
    dh = jnp.log(gh / eh) * 5.0

    fg = lab > 0
    zero = jnp.zeros_like(dx)
    wv = jnp.where(fg, zero + 1.0, zero)
    tgt_ref[0, 0, :] = jnp.where(fg, dx, zero)
    tgt_ref[0, 1, :] = jnp.where(fg, dy, zero)
    tgt_ref[0, 2, :] = jnp.where(fg, dw, zero)
    tgt_ref[0, 3, :] = jnp.where(fg, dh, zero)
    w_ref[0, 0, :] = wv
    w_ref[0, 1, :] = wv
    w_ref[0, 2, :] = wv
    w_ref[0, 3, :] = wv
    w2_ref[0, 0, :] = wv
    w2_ref[0, 1, :] = wv
    w2_ref[0, 2, :] = wv
    w2_ref[0, 3, :] = wv


_out_bs = pl.BlockSpec((1, 4, BLK), lambda i: (0, 0, i))

_tc_call = pl.pallas_call(
    _tc_body,
    grid=(GRID,),
    in_specs=[
        pl.BlockSpec((5, 1, BLK), lambda i: (0, 0, i)),
        pl.BlockSpec((1, 4, BLK), lambda i: (0, 0, i)),
        pl.BlockSpec((BLK,), lambda i: (i,)),
    ],
    out_specs=[_out_bs, _out_bs, _out_bs],
    out_shape=(
        jax.ShapeDtypeStruct((1, 4, N), jnp.float32),
        jax.ShapeDtypeStruct((1, 4, N), jnp.float32),
        jax.ShapeDtypeStruct((1, 4, N), jnp.float32),
    ),
)


@jax.jit
def kernel(gt_rois, rois, labels):
    rois_t = jnp.transpose(rois, (2, 0, 1))
    gt_t = jnp.transpose(gt_rois, (0, 2, 1))
    tgt_t, w_t, w2_t = _tc_call(rois_t, gt_t, labels)
    tgt = jnp.transpose(tgt_t, (0, 2, 1))
    w = jnp.transpose(w_t, (0, 2, 1))
    w2 = jnp.transpose(w2_t, (0, 2, 1))
    return tgt, w, w2
